# Initial kernel scaffold; baseline (speedup 1.0000x reference)
import functools

import jax
import jax.numpy as jnp
from jax import lax
from jax.experimental import pallas as pl
from jax.experimental.pallas import tpu as pltpu
from jax.experimental.pallas import tpu_sc as plsc

B, N, E, D, DE, H, HID2, ZD = 8, 1024, 16384, 128, 16, 128, 256, 12

_NC, _NS, _L = 2, 16, 16
_NW = _NC * _NS
_CH = 128


_RPW = (2 * B * E) // _NW


def _gather_body(table, idx_all, out, ibuf, rbuf, sem):
    c = lax.axis_index("c")
    s = lax.axis_index("s")
    wid = s * _NC + c
    base = wid * _RPW
    boff = ((wid * _RPW) // E % B) * N

    def step(t, carry):
        r0 = base + t * _CH
        pltpu.sync_copy(idx_all.at[pl.ds(r0, _CH)], ibuf)

        def addoff(k, carry2):
            ibuf[pl.ds(k * _L, _L)] = ibuf[pl.ds(k * _L, _L)] + boff
            return carry2

        lax.fori_loop(0, _CH // _L, addoff, 0)
        pltpu.async_copy(table.at[ibuf], rbuf, sem).wait()
        pltpu.sync_copy(rbuf, out.at[pl.ds(r0, _CH)])
        return carry

    lax.fori_loop(0, _RPW // _CH, step, 0)


def _sc_gather(table, idx_all):
    mesh = plsc.VectorSubcoreMesh(core_axis_name="c", subcore_axis_name="s")
    return pl.kernel(
        _gather_body,
        out_type=jax.ShapeDtypeStruct((2 * B * E, H), jnp.float32),
        mesh=mesh,
        scratch_types=[
            pltpu.VMEM((_CH,), jnp.int32),
            pltpu.VMEM((_CH, H), jnp.float32),
            pltpu.SemaphoreType.DMA,
        ],
    )(table, idx_all)


_BPC = B // _NC
_EPS = E // _NS
_ZROWS = _BPC * N // _NS


def _scatter_body(m, dst_all, out, shared, ibuf, mbuf, zbuf):
    c = lax.axis_index("c")
    s = lax.axis_index("s")

    def z1(i, carry):
        for k in range(H // _L):
            zbuf[i, pl.ds(k * _L, _L)] = jnp.zeros((_L,), jnp.float32)
        return carry

    lax.fori_loop(0, _CH, z1, 0)
    for r in range(_ZROWS // _CH):
        pltpu.sync_copy(zbuf, shared.at[pl.ds(s * _ZROWS + r * _CH, _CH)])
    plsc.subcore_barrier()

    def per_lb(lb, carry):
        def per_ch(ch, carry2):
            r0 = (c * _BPC + lb) * E + s * _EPS + ch * _CH
            pltpu.sync_copy(m.at[pl.ds(r0, _CH)], mbuf)
            pltpu.sync_copy(dst_all.at[pl.ds(r0, _CH)], ibuf)

            def addoff(k, carry3):
                ibuf[pl.ds(k * _L, _L)] = ibuf[pl.ds(k * _L, _L)] + lb * N
                return carry3

            lax.fori_loop(0, _CH // _L, addoff, 0)
            pltpu.sync_copy(mbuf, shared.at[ibuf], add=True)
            return carry2

        lax.fori_loop(0, _EPS // _CH, per_ch, 0)
        return carry

    lax.fori_loop(0, _BPC, per_lb, 0)
    plsc.subcore_barrier()

    pltpu.sync_copy(
        shared.at[pl.ds(s * _ZROWS, _ZROWS)],
        out.at[pl.ds(c * _BPC * N + s * _ZROWS, _ZROWS)],
    )


def _sc_scatter_add(m, dst_all):
    mesh = plsc.VectorSubcoreMesh(core_axis_name="c", subcore_axis_name="s")
    return pl.kernel(
        _scatter_body,
        out_type=jax.ShapeDtypeStruct((B * N, H), jnp.float32),
        mesh=mesh,
        scratch_types=[
            pltpu.VMEM_SHARED((_BPC * N, H), jnp.float32),
            pltpu.VMEM((_CH,), jnp.int32),
            pltpu.VMEM((_CH, H), jnp.float32),
            pltpu.VMEM((_CH, H), jnp.float32),
        ],
    )(m, dst_all)


_EBLK = 2048


def _edge_mlp_body(src, dst, ef, wa, wb, we, b1, w2, b2, out):
    h = (
        jnp.dot(src[...], wa[...], preferred_element_type=jnp.float32)
        + jnp.dot(dst[...], wb[...], preferred_element_type=jnp.float32)
        + jnp.dot(ef[...], we[...], preferred_element_type=jnp.float32)
        + b1[...]
    )
    h = jnp.maximum(h, 0.0)
    h = jnp.dot(h, w2[...], preferred_element_type=jnp.float32) + b2[...]
    out[...] = jnp.maximum(h, 0.0)


def _tc_edge_mlp(src, dst, ef, wa, wb, we, b1, w2, b2):
    rows = src.shape[0]
    grid = rows // _EBLK
    full = lambda shape: pl.BlockSpec(shape, lambda i: (0, 0))
    return pl.pallas_call(
        _edge_mlp_body,
        grid=(grid,),
        in_specs=[
            pl.BlockSpec((_EBLK, H), lambda i: (i, 0)),
            pl.BlockSpec((_EBLK, H), lambda i: (i, 0)),
            pl.BlockSpec((_EBLK, DE), lambda i: (i, 0)),
            full((H, H)),
            full((H, H)),
            full((DE, H)),
            full((1, H)),
            full((H, H)),
            full((1, H)),
        ],
        out_specs=pl.BlockSpec((_EBLK, H), lambda i: (i, 0)),
        out_shape=jax.ShapeDtypeStruct((rows, H), jnp.float32),
    )(src, dst, ef, wa, wb, we, b1, w2, b2)


_NBLK = 2048


def _node_mlp_body(nf, agg, prev, wa, wb, b1, w2, b2, out, *, add_prev):
    h = (
        jnp.dot(nf[...], wa[...], preferred_element_type=jnp.float32)
        + jnp.dot(agg[...], wb[...], preferred_element_type=jnp.float32)
        + b1[...]
    )
    h = jnp.maximum(h, 0.0)
    h = jnp.dot(h, w2[...], preferred_element_type=jnp.float32) + b2[...]
    h = jnp.where(h > 0.0, h, 0.2 * h)
    if add_prev:
        h = h + prev[...]
    out[...] = h


def _tc_node_mlp(nf, agg, prev, wa, wb, b1, w2, b2, add_prev):
    rows = nf.shape[0]
    grid = rows // _NBLK
    full = lambda shape: pl.BlockSpec(shape, lambda i: (0, 0))
    return pl.pallas_call(
        functools.partial(_node_mlp_body, add_prev=add_prev),
        grid=(grid,),
        in_specs=[
            pl.BlockSpec((_NBLK, H), lambda i: (i, 0)),
            pl.BlockSpec((_NBLK, H), lambda i: (i, 0)),
            pl.BlockSpec((_NBLK, H), lambda i: (i, 0)),
            full((H, H)),
            full((H, H)),
            full((1, H)),
            full((H, H)),
            full((1, H)),
        ],
        out_specs=pl.BlockSpec((_NBLK, H), lambda i: (i, 0)),
        out_shape=jax.ShapeDtypeStruct((rows, H), jnp.float32),
    )(nf, agg, prev, wa, wb, b1, w2, b2)


def _attn_body(nf, scale, ws, bs, wh, bh, wzm, bzm, wlv, blv, zm, lv):
    x = nf[0]
    sc = scale[0, 0]
    scores = sc * lax.dot_general(
        x, x, (((1,), (1,)), ((), ())), preferred_element_type=jnp.float32
    )
    rowmax = jnp.max(scores, axis=1, keepdims=True)
    ex = jnp.exp(scores - rowmax)
    rsum = jnp.sum(ex, axis=1, keepdims=True)
    colw = jnp.sum(ex / rsum, axis=0, keepdims=True) * (1.0 / N)
    pooled = jnp.dot(colw, x, preferred_element_type=jnp.float32)
    x1 = jnp.dot(pooled, ws[...], preferred_element_type=jnp.float32) + bs[...]
    x2 = jnp.dot(x1, wh[...], preferred_element_type=jnp.float32) + bh[...]
    x2 = jnp.maximum(x2, 0.0)
    zm[...] = jnp.dot(x2, wzm[...], preferred_element_type=jnp.float32) + bzm[...]
    lv[...] = jnp.dot(x2, wlv[...], preferred_element_type=jnp.float32) + blv[...]


def _tc_attn_head(nf, scale, ws, bs, wh, bh, wzm, bzm, wlv, blv):
    full = lambda shape: pl.BlockSpec(shape, lambda b: (0, 0))
    return pl.pallas_call(
        _attn_body,
        grid=(B,),
        in_specs=[
            pl.BlockSpec((1, N, H), lambda b: (b, 0, 0)),
            pl.BlockSpec(memory_space=pltpu.SMEM),
            full((H, H)),
            full((1, H)),
            full((H, HID2)),
            full((1, HID2)),
            full((HID2, H)),
            full((1, H)),
            full((HID2, H)),
            full((1, H)),
        ],
        out_specs=[
            pl.BlockSpec((1, H), lambda b: (b, 0)),
            pl.BlockSpec((1, H), lambda b: (b, 0)),
        ],
        out_shape=[
            jax.ShapeDtypeStruct((B, H), jnp.float32),
            jax.ShapeDtypeStruct((B, H), jnp.float32),
        ],
    )(nf, scale, ws, bs, wh, bh, wzm, bzm, wlv, blv)


def kernel(node_features, edge_features, edge_src, edge_dst, prop, params):
    nf = node_features.reshape(B * N, H)
    ef = edge_features.reshape(B * E, DE)
    idx_all = jnp.concatenate([edge_src.reshape(-1), edge_dst.reshape(-1)])
    dst_all = edge_dst.reshape(-1)

    prev = None
    for l in range(3):
        p = params["mp"][l]
        wa = p["Wm1"][:H]
        wb = p["Wm1"][H : 2 * H]
        we = p["Wm1"][2 * H :]
        gathered = _sc_gather(nf, idx_all)
        src_f = gathered[: B * E]
        dst_f = gathered[B * E :]
        m = _tc_edge_mlp(
            src_f, dst_f, ef, wa, wb, we,
            p["bm1"].reshape(1, H), p["Wm2"], p["bm2"].reshape(1, H),
        )
        agg = _sc_scatter_add(m, dst_all)
        nf_new = _tc_node_mlp(
            nf, agg, prev if prev is not None else nf,
            p["Wu1"][:H], p["Wu1"][H:], p["bu1"].reshape(1, H),
            p["Wu2"], p["bu2"].reshape(1, H),
            add_prev=prev is not None,
        )
        prev = nf_new
        nf = nf_new

    scale = params["attn_scale"].reshape(1, 1)
    wzm = jnp.pad(params["Wzm"], ((0, 0), (0, H - ZD)))
    bzm = jnp.pad(params["bzm"], (0, H - ZD)).reshape(1, H)
    wlv = jnp.pad(params["Wlv"], ((0, 0), (0, H - ZD)))
    blv = jnp.pad(params["blv"], (0, H - ZD)).reshape(1, H)
    zm_pad, lv_pad = _tc_attn_head(
        nf.reshape(B, N, H), scale,
        params["Ws2s"], params["bs2s"].reshape(1, H),
        params["Wh"], params["bh"].reshape(1, HID2),
        wzm, bzm, wlv, blv,
    )
    return (zm_pad[:, :ZD], lv_pad[:, :ZD])


# speedup vs baseline: 8.0414x; 8.0414x over previous
import functools

import jax
import jax.numpy as jnp
from jax import lax
from jax.experimental import pallas as pl
from jax.experimental.pallas import tpu as pltpu
from jax.experimental.pallas import tpu_sc as plsc

B, N, E, D, DE, H, HID2, ZD = 8, 1024, 16384, 128, 16, 128, 256, 12

_NC, _NS, _L = 2, 16, 16
_NW = _NC * _NS
_CH = 128


_RPW = (2 * B * E) // _NW


def _gather_body(table, idx_all, out, ibuf, rbuf, sem):
    c = lax.axis_index("c")
    s = lax.axis_index("s")
    wid = s * _NC + c
    base = wid * _RPW
    boff = ((wid * _RPW) // E % B) * N

    def step(t, carry):
        r0 = base + t * _CH
        pltpu.sync_copy(idx_all.at[pl.ds(r0, _CH)], ibuf)

        def addoff(k, carry2):
            ibuf[pl.ds(k * _L, _L)] = ibuf[pl.ds(k * _L, _L)] + boff
            return carry2

        lax.fori_loop(0, _CH // _L, addoff, 0)
        pltpu.async_copy(table.at[ibuf], rbuf, sem).wait()
        pltpu.sync_copy(rbuf, out.at[pl.ds(r0, _CH)])
        return carry

    lax.fori_loop(0, _RPW // _CH, step, 0)


def _sc_gather(table, idx_all):
    mesh = plsc.VectorSubcoreMesh(core_axis_name="c", subcore_axis_name="s", num_cores=_NC, num_subcores=_NS)
    return pl.kernel(
        _gather_body,
        out_type=jax.ShapeDtypeStruct((2 * B * E, H), jnp.float32),
        mesh=mesh,
        scratch_types=[
            pltpu.VMEM((_CH,), jnp.int32),
            pltpu.VMEM((_CH, H), jnp.float32),
            pltpu.SemaphoreType.DMA,
        ],
    )(table, idx_all)


_BPC = B // _NC
_EPS = E // _NS
_ZROWS = _BPC * N // _NS


def _scatter_body(m, dst_all, out, shared, ibuf, mbuf, zbuf):
    c = lax.axis_index("c")
    s = lax.axis_index("s")

    def z1(i, carry):
        for k in range(H // _L):
            zbuf[i, pl.ds(k * _L, _L)] = jnp.zeros((_L,), jnp.float32)
        return carry

    lax.fori_loop(0, _CH, z1, 0)
    for r in range(_ZROWS // _CH):
        pltpu.sync_copy(zbuf, shared.at[pl.ds(s * _ZROWS + r * _CH, _CH)])
    plsc.subcore_barrier()

    def per_lb(lb, carry):
        def per_ch(ch, carry2):
            r0 = (c * _BPC + lb) * E + s * _EPS + ch * _CH
            pltpu.sync_copy(m.at[pl.ds(r0, _CH)], mbuf)
            pltpu.sync_copy(dst_all.at[pl.ds(r0, _CH)], ibuf)

            def addoff(k, carry3):
                ibuf[pl.ds(k * _L, _L)] = ibuf[pl.ds(k * _L, _L)] + lb * N
                return carry3

            lax.fori_loop(0, _CH // _L, addoff, 0)
            pltpu.sync_copy(mbuf, shared.at[ibuf], add=True)
            return carry2

        lax.fori_loop(0, _EPS // _CH, per_ch, 0)
        return carry

    lax.fori_loop(0, _BPC, per_lb, 0)
    plsc.subcore_barrier()

    pltpu.sync_copy(
        shared.at[pl.ds(s * _ZROWS, _ZROWS)],
        out.at[pl.ds(c * _BPC * N + s * _ZROWS, _ZROWS)],
    )


def _sc_scatter_add(m, dst_all):
    mesh = plsc.VectorSubcoreMesh(core_axis_name="c", subcore_axis_name="s", num_cores=_NC, num_subcores=_NS)
    return pl.kernel(
        _scatter_body,
        out_type=jax.ShapeDtypeStruct((B * N, H), jnp.float32),
        mesh=mesh,
        scratch_types=[
            pltpu.VMEM_SHARED((_BPC * N, H), jnp.float32),
            pltpu.VMEM((_CH,), jnp.int32),
            pltpu.VMEM((_CH, H), jnp.float32),
            pltpu.VMEM((_CH, H), jnp.float32),
        ],
    )(m, dst_all)


_EBLK = 2048


def _edge_mlp_body(src, dst, ef, wa, wb, we, b1, w2, b2, out):
    h = (
        jnp.dot(src[...], wa[...], preferred_element_type=jnp.float32)
        + jnp.dot(dst[...], wb[...], preferred_element_type=jnp.float32)
        + jnp.dot(ef[...], we[...], preferred_element_type=jnp.float32)
        + b1[...]
    )
    h = jnp.maximum(h, 0.0)
    h = jnp.dot(h, w2[...], preferred_element_type=jnp.float32) + b2[...]
    out[...] = jnp.maximum(h, 0.0)


def _tc_edge_mlp(src, dst, ef, wa, wb, we, b1, w2, b2):
    rows = src.shape[0]
    grid = rows // _EBLK
    full = lambda shape: pl.BlockSpec(shape, lambda i: (0, 0))
    return pl.pallas_call(
        _edge_mlp_body,
        grid=(grid,),
        in_specs=[
            pl.BlockSpec((_EBLK, H), lambda i: (i, 0)),
            pl.BlockSpec((_EBLK, H), lambda i: (i, 0)),
            pl.BlockSpec((_EBLK, DE), lambda i: (i, 0)),
            full((H, H)),
            full((H, H)),
            full((DE, H)),
            full((1, H)),
            full((H, H)),
            full((1, H)),
        ],
        out_specs=pl.BlockSpec((_EBLK, H), lambda i: (i, 0)),
        out_shape=jax.ShapeDtypeStruct((rows, H), jnp.float32),
    )(src, dst, ef, wa, wb, we, b1, w2, b2)


_NBLK = 2048


def _node_mlp_body(nf, agg, prev, wa, wb, b1, w2, b2, out, *, add_prev):
    h = (
        jnp.dot(nf[...], wa[...], preferred_element_type=jnp.float32)
        + jnp.dot(agg[...], wb[...], preferred_element_type=jnp.float32)
        + b1[...]
    )
    h = jnp.maximum(h, 0.0)
    h = jnp.dot(h, w2[...], preferred_element_type=jnp.float32) + b2[...]
    h = jnp.maximum(h, 0.0)
    if add_prev:
        h = h + prev[...]
    out[...] = h


def _tc_node_mlp(nf, agg, prev, wa, wb, b1, w2, b2, add_prev):
    rows = nf.shape[0]
    grid = rows // _NBLK
    full = lambda shape: pl.BlockSpec(shape, lambda i: (0, 0))
    return pl.pallas_call(
        functools.partial(_node_mlp_body, add_prev=add_prev),
        grid=(grid,),
        in_specs=[
            pl.BlockSpec((_NBLK, H), lambda i: (i, 0)),
            pl.BlockSpec((_NBLK, H), lambda i: (i, 0)),
            pl.BlockSpec((_NBLK, H), lambda i: (i, 0)),
            full((H, H)),
            full((H, H)),
            full((1, H)),
            full((H, H)),
            full((1, H)),
        ],
        out_specs=pl.BlockSpec((_NBLK, H), lambda i: (i, 0)),
        out_shape=jax.ShapeDtypeStruct((rows, H), jnp.float32),
    )(nf, agg, prev, wa, wb, b1, w2, b2)


def _attn_body(nf, scale, ws, bs, wh, bh, wzm, bzm, wlv, blv, zm, lv):
    sc = scale[0, 0]
    pooled_rows = []
    for b in range(B):
        x = nf[b]
        scores = sc * lax.dot_general(
            x, x, (((1,), (1,)), ((), ())), preferred_element_type=jnp.float32
        )
        rowmax = jnp.max(scores, axis=1, keepdims=True)
        ex = jnp.exp(scores - rowmax)
        rsum = jnp.sum(ex, axis=1, keepdims=True)
        colw = jnp.sum(ex / rsum, axis=0, keepdims=True) * (1.0 / N)
        pooled_rows.append(jnp.dot(colw, x, preferred_element_type=jnp.float32))
    pooled = jnp.concatenate(pooled_rows, axis=0)
    x1 = jnp.dot(pooled, ws[...], preferred_element_type=jnp.float32) + bs[...]
    x2 = jnp.dot(x1, wh[...], preferred_element_type=jnp.float32) + bh[...]
    x2 = jnp.maximum(x2, 0.0)
    zm[...] = jnp.dot(x2, wzm[...], preferred_element_type=jnp.float32) + bzm[...]
    lv[...] = jnp.dot(x2, wlv[...], preferred_element_type=jnp.float32) + blv[...]


def _tc_attn_head(nf, scale, ws, bs, wh, bh, wzm, bzm, wlv, blv):
    return pl.pallas_call(
        _attn_body,
        in_specs=[
            pl.BlockSpec((B, N, H), lambda: (0, 0, 0)),
            pl.BlockSpec(memory_space=pltpu.SMEM),
            pl.BlockSpec((H, H), lambda: (0, 0)),
            pl.BlockSpec((1, H), lambda: (0, 0)),
            pl.BlockSpec((H, HID2), lambda: (0, 0)),
            pl.BlockSpec((1, HID2), lambda: (0, 0)),
            pl.BlockSpec((HID2, H), lambda: (0, 0)),
            pl.BlockSpec((1, H), lambda: (0, 0)),
            pl.BlockSpec((HID2, H), lambda: (0, 0)),
            pl.BlockSpec((1, H), lambda: (0, 0)),
        ],
        out_specs=[
            pl.BlockSpec((B, H), lambda: (0, 0)),
            pl.BlockSpec((B, H), lambda: (0, 0)),
        ],
        out_shape=[
            jax.ShapeDtypeStruct((B, H), jnp.float32),
            jax.ShapeDtypeStruct((B, H), jnp.float32),
        ],
    )(nf, scale, ws, bs, wh, bh, wzm, bzm, wlv, blv)


def kernel(node_features, edge_features, edge_src, edge_dst, prop, params):
    nf = node_features.reshape(B * N, H)
    ef = edge_features.reshape(B * E, DE)
    idx_all = jnp.concatenate([edge_src.reshape(-1), edge_dst.reshape(-1)])
    dst_all = edge_dst.reshape(-1)

    prev = None
    for l in range(3):
        p = params["mp"][l]
        wa = p["Wm1"][:H]
        wb = p["Wm1"][H : 2 * H]
        we = p["Wm1"][2 * H :]
        gathered = _sc_gather(nf, idx_all)
        src_f = gathered[: B * E]
        dst_f = gathered[B * E :]
        m = _tc_edge_mlp(
            src_f, dst_f, ef, wa, wb, we,
            p["bm1"].reshape(1, H), p["Wm2"], p["bm2"].reshape(1, H),
        )
        agg = _sc_scatter_add(m, dst_all)
        nf_new = _tc_node_mlp(
            nf, agg, prev if prev is not None else nf,
            p["Wu1"][:H], p["Wu1"][H:], p["bu1"].reshape(1, H),
            p["Wu2"], p["bu2"].reshape(1, H),
            add_prev=prev is not None,
        )
        prev = nf_new
        nf = nf_new

    scale = params["attn_scale"].reshape(1, 1)
    wzm = jnp.pad(params["Wzm"], ((0, 0), (0, H - ZD)))
    bzm = jnp.pad(params["bzm"], (0, H - ZD)).reshape(1, H)
    wlv = jnp.pad(params["Wlv"], ((0, 0), (0, H - ZD)))
    blv = jnp.pad(params["blv"], (0, H - ZD)).reshape(1, H)
    zm_pad, lv_pad = _tc_attn_head(
        nf.reshape(B, N, H), scale,
        params["Ws2s"], params["bs2s"].reshape(1, H),
        params["Wh"], params["bh"].reshape(1, HID2),
        wzm, bzm, wlv, blv,
    )
    return (zm_pad[:, :ZD], lv_pad[:, :ZD])


# speedup vs baseline: 12.9141x; 1.6059x over previous
import functools

import jax
import jax.numpy as jnp
from jax import lax
from jax.experimental import pallas as pl
from jax.experimental.pallas import tpu as pltpu
from jax.experimental.pallas import tpu_sc as plsc

B, N, E, D, DE, H, HID2, ZD = 8, 1024, 16384, 128, 16, 128, 256, 12

_NC, _NS, _L = 2, 16, 16
_NW = _NC * _NS
_CH = 128


_GRPW = (B * E) // (_NW // 2)
_GT = _GRPW // _CH


def _gather_run(table, idx, out, w, ia, ib, ra, rb, sga, sgb, swa, swb):
    base = w * _GRPW

    pltpu.sync_copy(idx.at[pl.ds(base, _CH)], ia)
    pltpu.async_copy(table.at[ia], ra, sga)

    def body(i, carry):
        r0 = base + (2 * i) * _CH
        r1 = r0 + _CH
        pltpu.sync_copy(idx.at[pl.ds(r1, _CH)], ib)
        pltpu.make_async_copy(table.at[ia], ra, sga).wait()

        @pl.when(i > 0)
        def _():
            pltpu.make_async_copy(rb, out.at[pl.ds(r1 - 2 * _CH, _CH)], swb).wait()

        pltpu.async_copy(table.at[ib], rb, sgb)
        pltpu.async_copy(ra, out.at[pl.ds(r0, _CH)], swa)

        @pl.when(i < _GT // 2 - 1)
        def _():
            pltpu.sync_copy(idx.at[pl.ds(r0 + 2 * _CH, _CH)], ia)

        pltpu.make_async_copy(table.at[ib], rb, sgb).wait()
        pltpu.make_async_copy(ra, out.at[pl.ds(r0, _CH)], swa).wait()

        @pl.when(i < _GT // 2 - 1)
        def _():
            pltpu.async_copy(table.at[ia], ra, sga)

        pltpu.async_copy(rb, out.at[pl.ds(r1, _CH)], swb)
        return carry

    lax.fori_loop(0, _GT // 2, body, 0)
    pltpu.make_async_copy(rb, out.at[pl.ds(base + (_GT - 1) * _CH, _CH)], swb).wait()


def _gather_body(table, sidx, didx, gs, gd, ia, ib, ra, rb, sga, sgb, swa, swb):
    c = lax.axis_index("c")
    s = lax.axis_index("s")
    wid = s * _NC + c

    @pl.when(wid < _NW // 2)
    def _():
        _gather_run(table, sidx, gs, wid, ia, ib, ra, rb, sga, sgb, swa, swb)

    @pl.when(wid >= _NW // 2)
    def _():
        _gather_run(table, didx, gd, wid - _NW // 2, ia, ib, ra, rb, sga, sgb, swa, swb)


def _sc_gather(table, sidx, didx):
    mesh = plsc.VectorSubcoreMesh(core_axis_name="c", subcore_axis_name="s", num_cores=_NC, num_subcores=_NS)
    return pl.kernel(
        _gather_body,
        out_type=[
            jax.ShapeDtypeStruct((B * E, H), jnp.float32),
            jax.ShapeDtypeStruct((B * E, H), jnp.float32),
        ],
        mesh=mesh,
        scratch_types=[
            pltpu.VMEM((_CH,), jnp.int32),
            pltpu.VMEM((_CH,), jnp.int32),
            pltpu.VMEM((_CH, H), jnp.float32),
            pltpu.VMEM((_CH, H), jnp.float32),
            pltpu.SemaphoreType.DMA,
            pltpu.SemaphoreType.DMA,
            pltpu.SemaphoreType.DMA,
            pltpu.SemaphoreType.DMA,
        ],
    )(table, sidx, didx)


_BPC = B // _NC
_EPS = E // _NS
_ST = _BPC * _EPS // _CH
_ZROWS = _BPC * N // _NS


def _scatter_body(m, dstl, out, shared, ia, ib, ma, mb, zbuf, sma, smb, ssa, ssb):
    c = lax.axis_index("c")
    s = lax.axis_index("s")

    def z1(i, carry):
        for k in range(H // _L):
            zbuf[i, pl.ds(k * _L, _L)] = jnp.zeros((_L,), jnp.float32)
        return carry

    lax.fori_loop(0, _CH, z1, 0)
    for r in range(_ZROWS // _CH):
        pltpu.sync_copy(zbuf, shared.at[pl.ds(s * _ZROWS + r * _CH, _CH)])
    plsc.subcore_barrier()

    def row(t):
        return (c * _BPC + t // (_EPS // _CH)) * E + s * _EPS + (t % (_EPS // _CH)) * _CH

    pltpu.async_copy(m.at[pl.ds(row(0), _CH)], ma, sma)
    pltpu.sync_copy(dstl.at[pl.ds(row(0), _CH)], ia)

    def body(i, carry):
        t0 = 2 * i
        r0 = row(t0)
        r1 = row(t0 + 1)

        @pl.when(i > 0)
        def _():
            pltpu.make_async_copy(mb, shared.at[ib], ssb).wait()

        pltpu.async_copy(m.at[pl.ds(r1, _CH)], mb, smb)
        pltpu.sync_copy(dstl.at[pl.ds(r1, _CH)], ib)
        pltpu.make_async_copy(m.at[pl.ds(r0, _CH)], ma, sma).wait()
        pltpu.async_copy(ma, shared.at[ia], ssa, add=True)
        pltpu.make_async_copy(ma, shared.at[ia], ssa).wait()

        @pl.when(i < _ST // 2 - 1)
        def _():
            r2 = row(t0 + 2)
            pltpu.async_copy(m.at[pl.ds(r2, _CH)], ma, sma)
            pltpu.sync_copy(dstl.at[pl.ds(r2, _CH)], ia)

        pltpu.make_async_copy(m.at[pl.ds(r1, _CH)], mb, smb).wait()
        pltpu.async_copy(mb, shared.at[ib], ssb, add=True)
        return carry

    lax.fori_loop(0, _ST // 2, body, 0)
    pltpu.make_async_copy(mb, shared.at[ib], ssb).wait()
    plsc.subcore_barrier()

    pltpu.sync_copy(
        shared.at[pl.ds(s * _ZROWS, _ZROWS)],
        out.at[pl.ds(c * _BPC * N + s * _ZROWS, _ZROWS)],
    )


def _sc_scatter_add(m, dstl):
    mesh = plsc.VectorSubcoreMesh(core_axis_name="c", subcore_axis_name="s", num_cores=_NC, num_subcores=_NS)
    return pl.kernel(
        _scatter_body,
        out_type=jax.ShapeDtypeStruct((B * N, H), jnp.float32),
        mesh=mesh,
        scratch_types=[
            pltpu.VMEM_SHARED((_BPC * N, H), jnp.float32),
            pltpu.VMEM((_CH,), jnp.int32),
            pltpu.VMEM((_CH,), jnp.int32),
            pltpu.VMEM((_CH, H), jnp.float32),
            pltpu.VMEM((_CH, H), jnp.float32),
            pltpu.VMEM((_CH, H), jnp.float32),
            pltpu.SemaphoreType.DMA,
            pltpu.SemaphoreType.DMA,
            pltpu.SemaphoreType.DMA,
            pltpu.SemaphoreType.DMA,
        ],
    )(m, dstl)


def _proj_body(nf, w, out):
    out[...] = jnp.dot(nf[...], w[0], preferred_element_type=jnp.float32)


def _tc_proj(nf, wstack):
    return pl.pallas_call(
        _proj_body,
        grid=(2,),
        in_specs=[
            pl.BlockSpec((B * N, H), lambda i: (0, 0)),
            pl.BlockSpec((1, H, H), lambda i: (i, 0, 0)),
        ],
        out_specs=pl.BlockSpec((B * N, H), lambda i: (i, 0)),
        out_shape=jax.ShapeDtypeStruct((2 * B * N, H), jnp.float32),
    )(nf, wstack)


_EBLK = 2048


def _edge_mlp_body(gs, gd, ef, we, b1, w2, b2, out):
    h = (
        gs[...]
        + gd[...]
        + jnp.dot(ef[...], we[...], preferred_element_type=jnp.float32)
        + b1[...]
    )
    h = jnp.maximum(h, 0.0)
    h = jnp.dot(h, w2[...], preferred_element_type=jnp.float32) + b2[...]
    out[...] = jnp.maximum(h, 0.0)


def _tc_edge_mlp(gs, gd, ef, we, b1, w2, b2):
    rows = gs.shape[0]
    grid = rows // _EBLK
    full = lambda shape: pl.BlockSpec(shape, lambda i: (0, 0))
    return pl.pallas_call(
        _edge_mlp_body,
        grid=(grid,),
        in_specs=[
            pl.BlockSpec((_EBLK, H), lambda i: (i, 0)),
            pl.BlockSpec((_EBLK, H), lambda i: (i, 0)),
            pl.BlockSpec((_EBLK, DE), lambda i: (i, 0)),
            full((DE, H)),
            full((1, H)),
            full((H, H)),
            full((1, H)),
        ],
        out_specs=pl.BlockSpec((_EBLK, H), lambda i: (i, 0)),
        out_shape=jax.ShapeDtypeStruct((rows, H), jnp.float32),
    )(gs, gd, ef, we, b1, w2, b2)


_NBLK = 2048


def _node_mlp_body(nf, agg, prev, wa, wb, b1, w2, b2, out, *, add_prev):
    h = (
        jnp.dot(nf[...], wa[...], preferred_element_type=jnp.float32)
        + jnp.dot(agg[...], wb[...], preferred_element_type=jnp.float32)
        + b1[...]
    )
    h = jnp.maximum(h, 0.0)
    h = jnp.dot(h, w2[...], preferred_element_type=jnp.float32) + b2[...]
    h = jnp.maximum(h, 0.0)
    if add_prev:
        h = h + prev[...]
    out[...] = h


def _tc_node_mlp(nf, agg, prev, wa, wb, b1, w2, b2, add_prev):
    rows = nf.shape[0]
    grid = rows // _NBLK
    full = lambda shape: pl.BlockSpec(shape, lambda i: (0, 0))
    return pl.pallas_call(
        functools.partial(_node_mlp_body, add_prev=add_prev),
        grid=(grid,),
        in_specs=[
            pl.BlockSpec((_NBLK, H), lambda i: (i, 0)),
            pl.BlockSpec((_NBLK, H), lambda i: (i, 0)),
            pl.BlockSpec((_NBLK, H), lambda i: (i, 0)),
            full((H, H)),
            full((H, H)),
            full((1, H)),
            full((H, H)),
            full((1, H)),
        ],
        out_specs=pl.BlockSpec((_NBLK, H), lambda i: (i, 0)),
        out_shape=jax.ShapeDtypeStruct((rows, H), jnp.float32),
    )(nf, agg, prev, wa, wb, b1, w2, b2)


def _attn_body(nf, scale, ws, bs, wh, bh, wzm, bzm, wlv, blv, zm, lv):
    sc = scale[0, 0]
    pooled_rows = []
    for b in range(B):
        x = nf[b]
        scores = sc * lax.dot_general(
            x, x, (((1,), (1,)), ((), ())), preferred_element_type=jnp.float32
        )
        rowmax = jnp.max(scores, axis=1, keepdims=True)
        ex = jnp.exp(scores - rowmax)
        rsum = jnp.sum(ex, axis=1, keepdims=True)
        colw = jnp.sum(ex / rsum, axis=0, keepdims=True) * (1.0 / N)
        pooled_rows.append(jnp.dot(colw, x, preferred_element_type=jnp.float32))
    pooled = jnp.concatenate(pooled_rows, axis=0)
    x1 = jnp.dot(pooled, ws[...], preferred_element_type=jnp.float32) + bs[...]
    x2 = jnp.dot(x1, wh[...], preferred_element_type=jnp.float32) + bh[...]
    x2 = jnp.maximum(x2, 0.0)
    zm[...] = jnp.dot(x2, wzm[...], preferred_element_type=jnp.float32) + bzm[...]
    lv[...] = jnp.dot(x2, wlv[...], preferred_element_type=jnp.float32) + blv[...]


def _tc_attn_head(nf, scale, ws, bs, wh, bh, wzm, bzm, wlv, blv):
    return pl.pallas_call(
        _attn_body,
        in_specs=[
            pl.BlockSpec((B, N, H), lambda: (0, 0, 0)),
            pl.BlockSpec(memory_space=pltpu.SMEM),
            pl.BlockSpec((H, H), lambda: (0, 0)),
            pl.BlockSpec((1, H), lambda: (0, 0)),
            pl.BlockSpec((H, HID2), lambda: (0, 0)),
            pl.BlockSpec((1, HID2), lambda: (0, 0)),
            pl.BlockSpec((HID2, H), lambda: (0, 0)),
            pl.BlockSpec((1, H), lambda: (0, 0)),
            pl.BlockSpec((HID2, H), lambda: (0, 0)),
            pl.BlockSpec((1, H), lambda: (0, 0)),
        ],
        out_specs=[
            pl.BlockSpec((B, H), lambda: (0, 0)),
            pl.BlockSpec((B, H), lambda: (0, 0)),
        ],
        out_shape=[
            jax.ShapeDtypeStruct((B, H), jnp.float32),
            jax.ShapeDtypeStruct((B, H), jnp.float32),
        ],
    )(nf, scale, ws, bs, wh, bh, wzm, bzm, wlv, blv)


def kernel(node_features, edge_features, edge_src, edge_dst, prop, params):
    nf = node_features.reshape(B * N, H)
    ef = edge_features.reshape(B * E, DE)
    boff = (jnp.arange(B, dtype=jnp.int32) * N)[:, None]
    sidx = (edge_src + boff).reshape(-1)
    didx = (edge_dst + boff + B * N).reshape(-1)
    lboff = ((jnp.arange(B, dtype=jnp.int32) % _BPC) * N)[:, None]
    dstl = (edge_dst + lboff).reshape(-1)

    prev = None
    for l in range(3):
        p = params["mp"][l]
        wstack = jnp.stack([p["Wm1"][:H], p["Wm1"][H : 2 * H]])
        table = _tc_proj(nf, wstack)
        gs, gd = _sc_gather(table, sidx, didx)
        m = _tc_edge_mlp(
            gs, gd, ef,
            p["Wm1"][2 * H :], p["bm1"].reshape(1, H),
            p["Wm2"], p["bm2"].reshape(1, H),
        )
        agg = _sc_scatter_add(m, dstl)
        nf_new = _tc_node_mlp(
            nf, agg, prev if prev is not None else nf,
            p["Wu1"][:H], p["Wu1"][H:], p["bu1"].reshape(1, H),
            p["Wu2"], p["bu2"].reshape(1, H),
            add_prev=prev is not None,
        )
        prev = nf_new
        nf = nf_new

    scale = params["attn_scale"].reshape(1, 1)
    wzm = jnp.pad(params["Wzm"], ((0, 0), (0, H - ZD)))
    bzm = jnp.pad(params["bzm"], (0, H - ZD)).reshape(1, H)
    wlv = jnp.pad(params["Wlv"], ((0, 0), (0, H - ZD)))
    blv = jnp.pad(params["blv"], (0, H - ZD)).reshape(1, H)
    zm_pad, lv_pad = _tc_attn_head(
        nf.reshape(B, N, H), scale,
        params["Ws2s"], params["bs2s"].reshape(1, H),
        params["Wh"], params["bh"].reshape(1, HID2),
        wzm, bzm, wlv, blv,
    )
    return (zm_pad[:, :ZD], lv_pad[:, :ZD])


# speedup vs baseline: 14.2201x; 1.1011x over previous
import functools

import jax
import jax.numpy as jnp
from jax import lax
from jax.experimental import pallas as pl
from jax.experimental.pallas import tpu as pltpu
from jax.experimental.pallas import tpu_sc as plsc

B, N, E, D, DE, H, HID2, ZD = 8, 1024, 16384, 128, 16, 128, 256, 12

_NC, _NS, _L = 2, 16, 16
_NW = _NC * _NS
_CH = 128


_GRPW = (B * E) // _NW
_GT = _GRPW // _CH


def _gather_body(table, sidx, didx, out, isa, isb, ida, idb, ra, rb, sga, sgb, swa, swb):
    c = lax.axis_index("c")
    s = lax.axis_index("s")
    base = (s * _NC + c) * _GRPW

    pltpu.sync_copy(sidx.at[pl.ds(base, _CH)], isa)
    pltpu.sync_copy(didx.at[pl.ds(base, _CH)], ida)
    pltpu.async_copy(table.at[isa], ra, sga)

    def body(i, carry):
        r0 = base + (2 * i) * _CH
        r1 = r0 + _CH
        pltpu.sync_copy(sidx.at[pl.ds(r1, _CH)], isb)
        pltpu.sync_copy(didx.at[pl.ds(r1, _CH)], idb)
        pltpu.make_async_copy(table.at[isa], ra, sga).wait()
        pltpu.async_copy(table.at[ida], ra, sga, add=True)

        @pl.when(i > 0)
        def _():
            pltpu.make_async_copy(rb, out.at[pl.ds(r1 - 2 * _CH, _CH)], swb).wait()

        pltpu.make_async_copy(table.at[ida], ra, sga).wait()
        pltpu.async_copy(table.at[isb], rb, sgb)
        pltpu.async_copy(ra, out.at[pl.ds(r0, _CH)], swa)

        @pl.when(i < _GT // 2 - 1)
        def _():
            pltpu.sync_copy(sidx.at[pl.ds(r0 + 2 * _CH, _CH)], isa)
            pltpu.sync_copy(didx.at[pl.ds(r0 + 2 * _CH, _CH)], ida)

        pltpu.make_async_copy(table.at[isb], rb, sgb).wait()
        pltpu.async_copy(table.at[idb], rb, sgb, add=True)
        pltpu.make_async_copy(ra, out.at[pl.ds(r0, _CH)], swa).wait()

        @pl.when(i < _GT // 2 - 1)
        def _():
            pltpu.async_copy(table.at[isa], ra, sga)

        pltpu.make_async_copy(table.at[idb], rb, sgb).wait()
        pltpu.async_copy(rb, out.at[pl.ds(r1, _CH)], swb)
        return carry

    lax.fori_loop(0, _GT // 2, body, 0)
    pltpu.make_async_copy(rb, out.at[pl.ds(base + (_GT - 1) * _CH, _CH)], swb).wait()


def _sc_gather(table, sidx, didx):
    mesh = plsc.VectorSubcoreMesh(core_axis_name="c", subcore_axis_name="s", num_cores=_NC, num_subcores=_NS)
    return pl.kernel(
        _gather_body,
        out_type=jax.ShapeDtypeStruct((B * E, H), jnp.float32),
        mesh=mesh,
        scratch_types=[
            pltpu.VMEM((_CH,), jnp.int32),
            pltpu.VMEM((_CH,), jnp.int32),
            pltpu.VMEM((_CH,), jnp.int32),
            pltpu.VMEM((_CH,), jnp.int32),
            pltpu.VMEM((_CH, H), jnp.float32),
            pltpu.VMEM((_CH, H), jnp.float32),
            pltpu.SemaphoreType.DMA,
            pltpu.SemaphoreType.DMA,
            pltpu.SemaphoreType.DMA,
            pltpu.SemaphoreType.DMA,
        ],
    )(table, sidx, didx)


_BPC = B // _NC
_EPS = E // _NS
_ST = _BPC * _EPS // _CH
_ZROWS = _BPC * N // _NS


def _scatter_body(m, dstl, out, shared, ia, ib, ma, mb, zbuf, sma, smb, ssa, ssb):
    c = lax.axis_index("c")
    s = lax.axis_index("s")

    def z1(i, carry):
        for k in range(H // _L):
            zbuf[i, pl.ds(k * _L, _L)] = jnp.zeros((_L,), jnp.float32)
        return carry

    lax.fori_loop(0, _CH, z1, 0)
    for r in range(_ZROWS // _CH):
        pltpu.sync_copy(zbuf, shared.at[pl.ds(s * _ZROWS + r * _CH, _CH)])
    plsc.subcore_barrier()

    def row(t):
        return (c * _BPC + t // (_EPS // _CH)) * E + s * _EPS + (t % (_EPS // _CH)) * _CH

    pltpu.async_copy(m.at[pl.ds(row(0), _CH)], ma, sma)
    pltpu.sync_copy(dstl.at[pl.ds(row(0), _CH)], ia)

    def body(i, carry):
        t0 = 2 * i
        r0 = row(t0)
        r1 = row(t0 + 1)

        @pl.when(i > 0)
        def _():
            pltpu.make_async_copy(mb, shared.at[ib], ssb).wait()

        pltpu.async_copy(m.at[pl.ds(r1, _CH)], mb, smb)
        pltpu.sync_copy(dstl.at[pl.ds(r1, _CH)], ib)
        pltpu.make_async_copy(m.at[pl.ds(r0, _CH)], ma, sma).wait()
        pltpu.async_copy(ma, shared.at[ia], ssa, add=True)
        pltpu.make_async_copy(ma, shared.at[ia], ssa).wait()

        @pl.when(i < _ST // 2 - 1)
        def _():
            r2 = row(t0 + 2)
            pltpu.async_copy(m.at[pl.ds(r2, _CH)], ma, sma)
            pltpu.sync_copy(dstl.at[pl.ds(r2, _CH)], ia)

        pltpu.make_async_copy(m.at[pl.ds(r1, _CH)], mb, smb).wait()
        pltpu.async_copy(mb, shared.at[ib], ssb, add=True)
        return carry

    lax.fori_loop(0, _ST // 2, body, 0)
    pltpu.make_async_copy(mb, shared.at[ib], ssb).wait()
    plsc.subcore_barrier()

    pltpu.sync_copy(
        shared.at[pl.ds(s * _ZROWS, _ZROWS)],
        out.at[pl.ds(c * _BPC * N + s * _ZROWS, _ZROWS)],
    )


def _sc_scatter_add(m, dstl):
    mesh = plsc.VectorSubcoreMesh(core_axis_name="c", subcore_axis_name="s", num_cores=_NC, num_subcores=_NS)
    return pl.kernel(
        _scatter_body,
        out_type=jax.ShapeDtypeStruct((B * N, H), jnp.float32),
        mesh=mesh,
        scratch_types=[
            pltpu.VMEM_SHARED((_BPC * N, H), jnp.float32),
            pltpu.VMEM((_CH,), jnp.int32),
            pltpu.VMEM((_CH,), jnp.int32),
            pltpu.VMEM((_CH, H), jnp.float32),
            pltpu.VMEM((_CH, H), jnp.float32),
            pltpu.VMEM((_CH, H), jnp.float32),
            pltpu.SemaphoreType.DMA,
            pltpu.SemaphoreType.DMA,
            pltpu.SemaphoreType.DMA,
            pltpu.SemaphoreType.DMA,
        ],
    )(m, dstl)


def _proj_body(nf, w, out):
    out[...] = jnp.dot(nf[...], w[0], preferred_element_type=jnp.float32)


def _tc_proj(nf, wstack):
    return pl.pallas_call(
        _proj_body,
        grid=(2,),
        in_specs=[
            pl.BlockSpec((B * N, H), lambda i: (0, 0)),
            pl.BlockSpec((1, H, H), lambda i: (i, 0, 0)),
        ],
        out_specs=pl.BlockSpec((B * N, H), lambda i: (i, 0)),
        out_shape=jax.ShapeDtypeStruct((2 * B * N, H), jnp.float32),
    )(nf, wstack)


_EBLK = 2048


def _edge_mlp_body(gsd, ef, we, b1, w2, b2, out):
    h = (
        gsd[...]
        + jnp.dot(ef[...], we[...], preferred_element_type=jnp.float32)
        + b1[...]
    )
    h = jnp.maximum(h, 0.0)
    h = jnp.dot(h, w2[...], preferred_element_type=jnp.float32) + b2[...]
    out[...] = jnp.maximum(h, 0.0)


def _tc_edge_mlp(gsd, ef, we, b1, w2, b2):
    rows = gsd.shape[0]
    grid = rows // _EBLK
    full = lambda shape: pl.BlockSpec(shape, lambda i: (0, 0))
    return pl.pallas_call(
        _edge_mlp_body,
        grid=(grid,),
        in_specs=[
            pl.BlockSpec((_EBLK, H), lambda i: (i, 0)),
            pl.BlockSpec((_EBLK, DE), lambda i: (i, 0)),
            full((DE, H)),
            full((1, H)),
            full((H, H)),
            full((1, H)),
        ],
        out_specs=pl.BlockSpec((_EBLK, H), lambda i: (i, 0)),
        out_shape=jax.ShapeDtypeStruct((rows, H), jnp.float32),
    )(gsd, ef, we, b1, w2, b2)


_NBLK = 2048


def _node_mlp_body(nf, agg, prev, wa, wb, b1, w2, b2, out, *, add_prev):
    h = (
        jnp.dot(nf[...], wa[...], preferred_element_type=jnp.float32)
        + jnp.dot(agg[...], wb[...], preferred_element_type=jnp.float32)
        + b1[...]
    )
    h = jnp.maximum(h, 0.0)
    h = jnp.dot(h, w2[...], preferred_element_type=jnp.float32) + b2[...]
    h = jnp.maximum(h, 0.0)
    if add_prev:
        h = h + prev[...]
    out[...] = h


def _tc_node_mlp(nf, agg, prev, wa, wb, b1, w2, b2, add_prev):
    rows = nf.shape[0]
    grid = rows // _NBLK
    full = lambda shape: pl.BlockSpec(shape, lambda i: (0, 0))
    return pl.pallas_call(
        functools.partial(_node_mlp_body, add_prev=add_prev),
        grid=(grid,),
        in_specs=[
            pl.BlockSpec((_NBLK, H), lambda i: (i, 0)),
            pl.BlockSpec((_NBLK, H), lambda i: (i, 0)),
            pl.BlockSpec((_NBLK, H), lambda i: (i, 0)),
            full((H, H)),
            full((H, H)),
            full((1, H)),
            full((H, H)),
            full((1, H)),
        ],
        out_specs=pl.BlockSpec((_NBLK, H), lambda i: (i, 0)),
        out_shape=jax.ShapeDtypeStruct((rows, H), jnp.float32),
    )(nf, agg, prev, wa, wb, b1, w2, b2)


def _attn_body(nf, scale, ws, bs, wh, bh, wzm, bzm, wlv, blv, zm, lv):
    sc = scale[0, 0]
    pooled_rows = []
    for b in range(B):
        x = nf[b]
        scores = sc * lax.dot_general(
            x, x, (((1,), (1,)), ((), ())), preferred_element_type=jnp.float32
        )
        rowmax = jnp.max(scores, axis=1, keepdims=True)
        ex = jnp.exp(scores - rowmax)
        rsum = jnp.sum(ex, axis=1, keepdims=True)
        colw = jnp.sum(ex / rsum, axis=0, keepdims=True) * (1.0 / N)
        pooled_rows.append(jnp.dot(colw, x, preferred_element_type=jnp.float32))
    pooled = jnp.concatenate(pooled_rows, axis=0)
    x1 = jnp.dot(pooled, ws[...], preferred_element_type=jnp.float32) + bs[...]
    x2 = jnp.dot(x1, wh[...], preferred_element_type=jnp.float32) + bh[...]
    x2 = jnp.maximum(x2, 0.0)
    zm[...] = jnp.dot(x2, wzm[...], preferred_element_type=jnp.float32) + bzm[...]
    lv[...] = jnp.dot(x2, wlv[...], preferred_element_type=jnp.float32) + blv[...]


def _tc_attn_head(nf, scale, ws, bs, wh, bh, wzm, bzm, wlv, blv):
    return pl.pallas_call(
        _attn_body,
        in_specs=[
            pl.BlockSpec((B, N, H), lambda: (0, 0, 0)),
            pl.BlockSpec(memory_space=pltpu.SMEM),
            pl.BlockSpec((H, H), lambda: (0, 0)),
            pl.BlockSpec((1, H), lambda: (0, 0)),
            pl.BlockSpec((H, HID2), lambda: (0, 0)),
            pl.BlockSpec((1, HID2), lambda: (0, 0)),
            pl.BlockSpec((HID2, H), lambda: (0, 0)),
            pl.BlockSpec((1, H), lambda: (0, 0)),
            pl.BlockSpec((HID2, H), lambda: (0, 0)),
            pl.BlockSpec((1, H), lambda: (0, 0)),
        ],
        out_specs=[
            pl.BlockSpec((B, H), lambda: (0, 0)),
            pl.BlockSpec((B, H), lambda: (0, 0)),
        ],
        out_shape=[
            jax.ShapeDtypeStruct((B, H), jnp.float32),
            jax.ShapeDtypeStruct((B, H), jnp.float32),
        ],
    )(nf, scale, ws, bs, wh, bh, wzm, bzm, wlv, blv)


def kernel(node_features, edge_features, edge_src, edge_dst, prop, params):
    nf = node_features.reshape(B * N, H)
    ef = edge_features.reshape(B * E, DE)
    boff = (jnp.arange(B, dtype=jnp.int32) * N)[:, None]
    sidx = (edge_src + boff).reshape(-1)
    didx = (edge_dst + boff + B * N).reshape(-1)
    lboff = ((jnp.arange(B, dtype=jnp.int32) % _BPC) * N)[:, None]
    dstl = (edge_dst + lboff).reshape(-1)

    prev = None
    for l in range(3):
        p = params["mp"][l]
        wstack = jnp.stack([p["Wm1"][:H], p["Wm1"][H : 2 * H]])
        table = _tc_proj(nf, wstack)
        gsd = _sc_gather(table, sidx, didx)
        m = _tc_edge_mlp(
            gsd, ef,
            p["Wm1"][2 * H :], p["bm1"].reshape(1, H),
            p["Wm2"], p["bm2"].reshape(1, H),
        )
        agg = _sc_scatter_add(m, dstl)
        nf_new = _tc_node_mlp(
            nf, agg, prev if prev is not None else nf,
            p["Wu1"][:H], p["Wu1"][H:], p["bu1"].reshape(1, H),
            p["Wu2"], p["bu2"].reshape(1, H),
            add_prev=prev is not None,
        )
        prev = nf_new
        nf = nf_new

    scale = params["attn_scale"].reshape(1, 1)
    wzm = jnp.pad(params["Wzm"], ((0, 0), (0, H - ZD)))
    bzm = jnp.pad(params["bzm"], (0, H - ZD)).reshape(1, H)
    wlv = jnp.pad(params["Wlv"], ((0, 0), (0, H - ZD)))
    blv = jnp.pad(params["blv"], (0, H - ZD)).reshape(1, H)
    zm_pad, lv_pad = _tc_attn_head(
        nf.reshape(B, N, H), scale,
        params["Ws2s"], params["bs2s"].reshape(1, H),
        params["Wh"], params["bh"].reshape(1, HID2),
        wzm, bzm, wlv, blv,
    )
    return (zm_pad[:, :ZD], lv_pad[:, :ZD])


# speedup vs baseline: 15.4931x; 1.0895x over previous
import functools

import jax
import jax.numpy as jnp
from jax import lax
from jax.experimental import pallas as pl
from jax.experimental.pallas import tpu as pltpu
from jax.experimental.pallas import tpu_sc as plsc

B, N, E, D, DE, H, HID2, ZD = 8, 1024, 16384, 128, 16, 128, 256, 12

_NC, _NS, _L = 2, 16, 16
_NW = _NC * _NS
_CH = 128


def _gather_body(table, sidx, didx, out, isa, isb, ida, idb, ra, rb, sga, sgb, swa, swb, *, grpw, gt):
    c = lax.axis_index("c")
    s = lax.axis_index("s")
    base = (s * _NC + c) * grpw
    _GT = gt

    pltpu.sync_copy(sidx.at[pl.ds(base, _CH)], isa)
    pltpu.sync_copy(didx.at[pl.ds(base, _CH)], ida)
    pltpu.async_copy(table.at[isa], ra, sga)

    def body(i, carry):
        r0 = base + (2 * i) * _CH
        r1 = r0 + _CH
        pltpu.sync_copy(sidx.at[pl.ds(r1, _CH)], isb)
        pltpu.sync_copy(didx.at[pl.ds(r1, _CH)], idb)
        pltpu.make_async_copy(table.at[isa], ra, sga).wait()
        pltpu.async_copy(table.at[ida], ra, sga, add=True)

        @pl.when(i > 0)
        def _():
            pltpu.make_async_copy(rb, out.at[pl.ds(r1 - 2 * _CH, _CH)], swb).wait()

        pltpu.make_async_copy(table.at[ida], ra, sga).wait()
        pltpu.async_copy(table.at[isb], rb, sgb)
        pltpu.async_copy(ra, out.at[pl.ds(r0, _CH)], swa)

        @pl.when(i < _GT // 2 - 1)
        def _():
            pltpu.sync_copy(sidx.at[pl.ds(r0 + 2 * _CH, _CH)], isa)
            pltpu.sync_copy(didx.at[pl.ds(r0 + 2 * _CH, _CH)], ida)

        pltpu.make_async_copy(table.at[isb], rb, sgb).wait()
        pltpu.async_copy(table.at[idb], rb, sgb, add=True)
        pltpu.make_async_copy(ra, out.at[pl.ds(r0, _CH)], swa).wait()

        @pl.when(i < _GT // 2 - 1)
        def _():
            pltpu.async_copy(table.at[isa], ra, sga)

        pltpu.make_async_copy(table.at[idb], rb, sgb).wait()
        pltpu.async_copy(rb, out.at[pl.ds(r1, _CH)], swb)
        return carry

    lax.fori_loop(0, _GT // 2, body, 0)
    pltpu.make_async_copy(rb, out.at[pl.ds(base + (_GT - 1) * _CH, _CH)], swb).wait()


def _sc_gather(table, sidx, didx):
    rows = sidx.shape[0]
    grpw = rows // _NW
    mesh = plsc.VectorSubcoreMesh(core_axis_name="c", subcore_axis_name="s", num_cores=_NC, num_subcores=_NS)
    return pl.kernel(
        functools.partial(_gather_body, grpw=grpw, gt=grpw // _CH),
        out_type=jax.ShapeDtypeStruct((rows, H), jnp.float32),
        mesh=mesh,
        scratch_types=[
            pltpu.VMEM((_CH,), jnp.int32),
            pltpu.VMEM((_CH,), jnp.int32),
            pltpu.VMEM((_CH,), jnp.int32),
            pltpu.VMEM((_CH,), jnp.int32),
            pltpu.VMEM((_CH, H), jnp.float32),
            pltpu.VMEM((_CH, H), jnp.float32),
            pltpu.SemaphoreType.DMA,
            pltpu.SemaphoreType.DMA,
            pltpu.SemaphoreType.DMA,
            pltpu.SemaphoreType.DMA,
        ],
    )(table, sidx, didx)


_EPS = E // _NS


def _scatter_body(m, dstl, out, shared, ia, ib, ma, mb, zbuf, sma, smb, ssa, ssb, *, bpc):
    c = lax.axis_index("c")
    s = lax.axis_index("s")
    _BPC = bpc
    _ST = bpc * _EPS // _CH
    _ZROWS = bpc * N // _NS

    def z1(i, carry):
        for k in range(H // _L):
            zbuf[i, pl.ds(k * _L, _L)] = jnp.zeros((_L,), jnp.float32)
        return carry

    lax.fori_loop(0, _CH, z1, 0)
    for r in range(_ZROWS // _CH):
        pltpu.sync_copy(zbuf, shared.at[pl.ds(s * _ZROWS + r * _CH, _CH)])
    plsc.subcore_barrier()

    def row(t):
        return (c * _BPC + t // (_EPS // _CH)) * E + s * _EPS + (t % (_EPS // _CH)) * _CH

    pltpu.async_copy(m.at[pl.ds(row(0), _CH)], ma, sma)
    pltpu.sync_copy(dstl.at[pl.ds(row(0), _CH)], ia)

    def body(i, carry):
        t0 = 2 * i
        r0 = row(t0)
        r1 = row(t0 + 1)

        @pl.when(i > 0)
        def _():
            pltpu.make_async_copy(mb, shared.at[ib], ssb).wait()

        pltpu.async_copy(m.at[pl.ds(r1, _CH)], mb, smb)
        pltpu.sync_copy(dstl.at[pl.ds(r1, _CH)], ib)
        pltpu.make_async_copy(m.at[pl.ds(r0, _CH)], ma, sma).wait()
        pltpu.async_copy(ma, shared.at[ia], ssa, add=True)
        pltpu.make_async_copy(ma, shared.at[ia], ssa).wait()

        @pl.when(i < _ST // 2 - 1)
        def _():
            r2 = row(t0 + 2)
            pltpu.async_copy(m.at[pl.ds(r2, _CH)], ma, sma)
            pltpu.sync_copy(dstl.at[pl.ds(r2, _CH)], ia)

        pltpu.make_async_copy(m.at[pl.ds(r1, _CH)], mb, smb).wait()
        pltpu.async_copy(mb, shared.at[ib], ssb, add=True)
        return carry

    lax.fori_loop(0, _ST // 2, body, 0)
    pltpu.make_async_copy(mb, shared.at[ib], ssb).wait()
    plsc.subcore_barrier()

    pltpu.sync_copy(
        shared.at[pl.ds(s * _ZROWS, _ZROWS)],
        out.at[pl.ds(c * _BPC * N + s * _ZROWS, _ZROWS)],
    )


def _sc_scatter_add(m, dstl):
    nb = m.shape[0] // E
    bpc = nb // _NC
    mesh = plsc.VectorSubcoreMesh(core_axis_name="c", subcore_axis_name="s", num_cores=_NC, num_subcores=_NS)
    return pl.kernel(
        functools.partial(_scatter_body, bpc=bpc),
        out_type=jax.ShapeDtypeStruct((nb * N, H), jnp.float32),
        mesh=mesh,
        scratch_types=[
            pltpu.VMEM_SHARED((bpc * N, H), jnp.float32),
            pltpu.VMEM((_CH,), jnp.int32),
            pltpu.VMEM((_CH,), jnp.int32),
            pltpu.VMEM((_CH, H), jnp.float32),
            pltpu.VMEM((_CH, H), jnp.float32),
            pltpu.VMEM((_CH, H), jnp.float32),
            pltpu.SemaphoreType.DMA,
            pltpu.SemaphoreType.DMA,
            pltpu.SemaphoreType.DMA,
            pltpu.SemaphoreType.DMA,
        ],
    )(m, dstl)


def _proj_body(nf, w, out):
    out[...] = jnp.dot(nf[...], w[0], preferred_element_type=jnp.float32)


def _tc_proj(nf, wstack):
    return pl.pallas_call(
        _proj_body,
        grid=(2,),
        in_specs=[
            pl.BlockSpec((B * N, H), lambda i: (0, 0)),
            pl.BlockSpec((1, H, H), lambda i: (i, 0, 0)),
        ],
        out_specs=pl.BlockSpec((B * N, H), lambda i: (i, 0)),
        out_shape=jax.ShapeDtypeStruct((2 * B * N, H), jnp.float32),
    )(nf, wstack)


_EBLK = 2048


def _edge_mlp_body(gsd, ef, we, b1, w2, b2, out):
    h = (
        gsd[...]
        + jnp.dot(ef[...], we[...], preferred_element_type=jnp.float32)
        + b1[...]
    )
    h = jnp.maximum(h, 0.0)
    h = jnp.dot(h, w2[...], preferred_element_type=jnp.float32) + b2[...]
    out[...] = jnp.maximum(h, 0.0)


def _tc_edge_mlp(gsd, ef, we, b1, w2, b2):
    rows = gsd.shape[0]
    grid = rows // _EBLK
    full = lambda shape: pl.BlockSpec(shape, lambda i: (0, 0))
    return pl.pallas_call(
        _edge_mlp_body,
        grid=(grid,),
        in_specs=[
            pl.BlockSpec((_EBLK, H), lambda i: (i, 0)),
            pl.BlockSpec((_EBLK, DE), lambda i: (i, 0)),
            full((DE, H)),
            full((1, H)),
            full((H, H)),
            full((1, H)),
        ],
        out_specs=pl.BlockSpec((_EBLK, H), lambda i: (i, 0)),
        out_shape=jax.ShapeDtypeStruct((rows, H), jnp.float32),
    )(gsd, ef, we, b1, w2, b2)


_NBLK = 2048


def _node_mlp_body(nf, agg, prev, wa, wb, b1, w2, b2, out, *, add_prev):
    h = (
        jnp.dot(nf[...], wa[...], preferred_element_type=jnp.float32)
        + jnp.dot(agg[...], wb[...], preferred_element_type=jnp.float32)
        + b1[...]
    )
    h = jnp.maximum(h, 0.0)
    h = jnp.dot(h, w2[...], preferred_element_type=jnp.float32) + b2[...]
    h = jnp.maximum(h, 0.0)
    if add_prev:
        h = h + prev[...]
    out[...] = h


def _tc_node_mlp(nf, agg, prev, wa, wb, b1, w2, b2, add_prev):
    rows = nf.shape[0]
    grid = rows // _NBLK
    full = lambda shape: pl.BlockSpec(shape, lambda i: (0, 0))
    return pl.pallas_call(
        functools.partial(_node_mlp_body, add_prev=add_prev),
        grid=(grid,),
        in_specs=[
            pl.BlockSpec((_NBLK, H), lambda i: (i, 0)),
            pl.BlockSpec((_NBLK, H), lambda i: (i, 0)),
            pl.BlockSpec((_NBLK, H), lambda i: (i, 0)),
            full((H, H)),
            full((H, H)),
            full((1, H)),
            full((H, H)),
            full((1, H)),
        ],
        out_specs=pl.BlockSpec((_NBLK, H), lambda i: (i, 0)),
        out_shape=jax.ShapeDtypeStruct((rows, H), jnp.float32),
    )(nf, agg, prev, wa, wb, b1, w2, b2)


def _attn_body(nf, scale, ws, bs, wh, bh, wzm, bzm, wlv, blv, zm, lv):
    sc = scale[0, 0]
    pooled_rows = []
    for b in range(B):
        x = nf[b]
        scores = sc * lax.dot_general(
            x, x, (((1,), (1,)), ((), ())), preferred_element_type=jnp.float32
        )
        rowmax = jnp.max(scores, axis=1, keepdims=True)
        ex = jnp.exp(scores - rowmax)
        rsum = jnp.sum(ex, axis=1, keepdims=True)
        colw = jnp.sum(ex / rsum, axis=0, keepdims=True) * (1.0 / N)
        pooled_rows.append(jnp.dot(colw, x, preferred_element_type=jnp.float32))
    pooled = jnp.concatenate(pooled_rows, axis=0)
    x1 = jnp.dot(pooled, ws[...], preferred_element_type=jnp.float32) + bs[...]
    x2 = jnp.dot(x1, wh[...], preferred_element_type=jnp.float32) + bh[...]
    x2 = jnp.maximum(x2, 0.0)
    zm[...] = jnp.dot(x2, wzm[...], preferred_element_type=jnp.float32) + bzm[...]
    lv[...] = jnp.dot(x2, wlv[...], preferred_element_type=jnp.float32) + blv[...]


def _tc_attn_head(nf, scale, ws, bs, wh, bh, wzm, bzm, wlv, blv):
    return pl.pallas_call(
        _attn_body,
        in_specs=[
            pl.BlockSpec((B, N, H), lambda: (0, 0, 0)),
            pl.BlockSpec(memory_space=pltpu.SMEM),
            pl.BlockSpec((H, H), lambda: (0, 0)),
            pl.BlockSpec((1, H), lambda: (0, 0)),
            pl.BlockSpec((H, HID2), lambda: (0, 0)),
            pl.BlockSpec((1, HID2), lambda: (0, 0)),
            pl.BlockSpec((HID2, H), lambda: (0, 0)),
            pl.BlockSpec((1, H), lambda: (0, 0)),
            pl.BlockSpec((HID2, H), lambda: (0, 0)),
            pl.BlockSpec((1, H), lambda: (0, 0)),
        ],
        out_specs=[
            pl.BlockSpec((B, H), lambda: (0, 0)),
            pl.BlockSpec((B, H), lambda: (0, 0)),
        ],
        out_shape=[
            jax.ShapeDtypeStruct((B, H), jnp.float32),
            jax.ShapeDtypeStruct((B, H), jnp.float32),
        ],
    )(nf, scale, ws, bs, wh, bh, wzm, bzm, wlv, blv)


def kernel(node_features, edge_features, edge_src, edge_dst, prop, params):
    nf = node_features.reshape(B * N, H)
    ef = edge_features.reshape(B * E, DE)
    boff = (jnp.arange(B, dtype=jnp.int32) * N)[:, None]
    sidx = (edge_src + boff).reshape(-1)
    didx = (edge_dst + boff + B * N).reshape(-1)
    lboff = ((jnp.arange(B, dtype=jnp.int32) % 2) * N)[:, None]
    dstl = (edge_dst + lboff).reshape(-1)

    hE, hN = (B // 2) * E, (B // 2) * N
    sx = [sidx[h * hE : (h + 1) * hE] for h in range(2)]
    dx = [didx[h * hE : (h + 1) * hE] for h in range(2)]
    dl = [dstl[h * hE : (h + 1) * hE] for h in range(2)]
    efh = [ef[h * hE : (h + 1) * hE] for h in range(2)]

    prev = None
    for l in range(3):
        p = params["mp"][l]
        wstack = jnp.stack([p["Wm1"][:H], p["Wm1"][H : 2 * H]])
        we, bm1 = p["Wm1"][2 * H :], p["bm1"].reshape(1, H)
        wm2, bm2 = p["Wm2"], p["bm2"].reshape(1, H)
        table = _tc_proj(nf, wstack)
        g = [_sc_gather(table, sx[h], dx[h]) for h in range(2)]
        mA = _tc_edge_mlp(g[0], efh[0], we, bm1, wm2, bm2)
        aggA = _sc_scatter_add(mA, dl[0])
        mB = _tc_edge_mlp(g[1], efh[1], we, bm1, wm2, bm2)
        aggB = _sc_scatter_add(mB, dl[1])
        agg = [aggA, aggB]
        nf_halves = []
        for h in range(2):
            nfh = nf[h * hN : (h + 1) * hN]
            prevh = prev[h * hN : (h + 1) * hN] if prev is not None else nfh
            nf_halves.append(
                _tc_node_mlp(
                    nfh, agg[h], prevh,
                    p["Wu1"][:H], p["Wu1"][H:], p["bu1"].reshape(1, H),
                    p["Wu2"], p["bu2"].reshape(1, H),
                    add_prev=prev is not None,
                )
            )
        nf_new = jnp.concatenate(nf_halves, axis=0)
        prev = nf_new
        nf = nf_new

    scale = params["attn_scale"].reshape(1, 1)
    wzm = jnp.pad(params["Wzm"], ((0, 0), (0, H - ZD)))
    bzm = jnp.pad(params["bzm"], (0, H - ZD)).reshape(1, H)
    wlv = jnp.pad(params["Wlv"], ((0, 0), (0, H - ZD)))
    blv = jnp.pad(params["blv"], (0, H - ZD)).reshape(1, H)
    zm_pad, lv_pad = _tc_attn_head(
        nf.reshape(B, N, H), scale,
        params["Ws2s"], params["bs2s"].reshape(1, H),
        params["Wh"], params["bh"].reshape(1, HID2),
        wzm, bzm, wlv, blv,
    )
    return (zm_pad[:, :ZD], lv_pad[:, :ZD])


# speedup vs baseline: 15.8950x; 1.0259x over previous
import functools

import jax
import jax.numpy as jnp
from jax import lax
from jax.experimental import pallas as pl
from jax.experimental.pallas import tpu as pltpu
from jax.experimental.pallas import tpu_sc as plsc

B, N, E, D, DE, H, HID2, ZD = 8, 1024, 16384, 128, 16, 128, 256, 12

_NC, _NS, _L = 2, 16, 16
_NW = _NC * _NS
_CH = 128


def _gather_body(table, sidx, didx, out, isa, isb, ida, idb, ra, rb, sga, sgb, swa, swb, *, grpw, gt):
    c = lax.axis_index("c")
    s = lax.axis_index("s")
    base = (s * _NC + c) * grpw
    _GT = gt

    pltpu.sync_copy(sidx.at[pl.ds(base, _CH)], isa)
    pltpu.sync_copy(didx.at[pl.ds(base, _CH)], ida)
    pltpu.async_copy(table.at[isa], ra, sga)

    def body(i, carry):
        r0 = base + (2 * i) * _CH
        r1 = r0 + _CH
        pltpu.sync_copy(sidx.at[pl.ds(r1, _CH)], isb)
        pltpu.sync_copy(didx.at[pl.ds(r1, _CH)], idb)
        pltpu.make_async_copy(table.at[isa], ra, sga).wait()
        pltpu.async_copy(table.at[ida], ra, sga, add=True)

        @pl.when(i > 0)
        def _():
            pltpu.make_async_copy(rb, out.at[pl.ds(r1 - 2 * _CH, _CH)], swb).wait()

        pltpu.make_async_copy(table.at[ida], ra, sga).wait()
        pltpu.async_copy(table.at[isb], rb, sgb)
        pltpu.async_copy(ra, out.at[pl.ds(r0, _CH)], swa)

        @pl.when(i < _GT // 2 - 1)
        def _():
            pltpu.sync_copy(sidx.at[pl.ds(r0 + 2 * _CH, _CH)], isa)
            pltpu.sync_copy(didx.at[pl.ds(r0 + 2 * _CH, _CH)], ida)

        pltpu.make_async_copy(table.at[isb], rb, sgb).wait()
        pltpu.async_copy(table.at[idb], rb, sgb, add=True)
        pltpu.make_async_copy(ra, out.at[pl.ds(r0, _CH)], swa).wait()

        @pl.when(i < _GT // 2 - 1)
        def _():
            pltpu.async_copy(table.at[isa], ra, sga)

        pltpu.make_async_copy(table.at[idb], rb, sgb).wait()
        pltpu.async_copy(rb, out.at[pl.ds(r1, _CH)], swb)
        return carry

    lax.fori_loop(0, _GT // 2, body, 0)
    pltpu.make_async_copy(rb, out.at[pl.ds(base + (_GT - 1) * _CH, _CH)], swb).wait()


def _sc_gather(table, sidx, didx):
    rows = sidx.shape[0]
    grpw = rows // _NW
    mesh = plsc.VectorSubcoreMesh(core_axis_name="c", subcore_axis_name="s", num_cores=_NC, num_subcores=_NS)
    return pl.kernel(
        functools.partial(_gather_body, grpw=grpw, gt=grpw // _CH),
        out_type=jax.ShapeDtypeStruct((rows, H), jnp.float32),
        mesh=mesh,
        scratch_types=[
            pltpu.VMEM((_CH,), jnp.int32),
            pltpu.VMEM((_CH,), jnp.int32),
            pltpu.VMEM((_CH,), jnp.int32),
            pltpu.VMEM((_CH,), jnp.int32),
            pltpu.VMEM((_CH, H), jnp.float32),
            pltpu.VMEM((_CH, H), jnp.float32),
            pltpu.SemaphoreType.DMA,
            pltpu.SemaphoreType.DMA,
            pltpu.SemaphoreType.DMA,
            pltpu.SemaphoreType.DMA,
        ],
    )(table, sidx, didx)


_EPS = E // _NS


def _scatter_body(m, dstl, out, shared, ia, ib, ma, mb, zbuf, sma, smb, ssa, ssb, *, bpc):
    c = lax.axis_index("c")
    s = lax.axis_index("s")
    _BPC = bpc
    _ST = bpc * _EPS // _CH
    _ZROWS = bpc * N // _NS

    def z1(i, carry):
        for k in range(H // _L):
            zbuf[i, pl.ds(k * _L, _L)] = jnp.zeros((_L,), jnp.float32)
        return carry

    lax.fori_loop(0, _CH, z1, 0)
    for r in range(_ZROWS // _CH):
        pltpu.sync_copy(zbuf, shared.at[pl.ds(s * _ZROWS + r * _CH, _CH)])
    plsc.subcore_barrier()

    def row(t):
        return (c * _BPC + t // (_EPS // _CH)) * E + s * _EPS + (t % (_EPS // _CH)) * _CH

    pltpu.async_copy(m.at[pl.ds(row(0), _CH)], ma, sma)
    pltpu.sync_copy(dstl.at[pl.ds(row(0), _CH)], ia)

    def body(i, carry):
        t0 = 2 * i
        r0 = row(t0)
        r1 = row(t0 + 1)

        @pl.when(i > 0)
        def _():
            pltpu.make_async_copy(mb, shared.at[ib], ssb).wait()

        pltpu.async_copy(m.at[pl.ds(r1, _CH)], mb, smb)
        pltpu.sync_copy(dstl.at[pl.ds(r1, _CH)], ib)
        pltpu.make_async_copy(m.at[pl.ds(r0, _CH)], ma, sma).wait()
        pltpu.async_copy(ma, shared.at[ia], ssa, add=True)
        pltpu.make_async_copy(ma, shared.at[ia], ssa).wait()

        @pl.when(i < _ST // 2 - 1)
        def _():
            r2 = row(t0 + 2)
            pltpu.async_copy(m.at[pl.ds(r2, _CH)], ma, sma)
            pltpu.sync_copy(dstl.at[pl.ds(r2, _CH)], ia)

        pltpu.make_async_copy(m.at[pl.ds(r1, _CH)], mb, smb).wait()
        pltpu.async_copy(mb, shared.at[ib], ssb, add=True)
        return carry

    lax.fori_loop(0, _ST // 2, body, 0)
    pltpu.make_async_copy(mb, shared.at[ib], ssb).wait()
    plsc.subcore_barrier()

    pltpu.sync_copy(
        shared.at[pl.ds(s * _ZROWS, _ZROWS)],
        out.at[pl.ds(c * _BPC * N + s * _ZROWS, _ZROWS)],
    )


def _sc_scatter_add(m, dstl):
    nb = m.shape[0] // E
    bpc = nb // _NC
    mesh = plsc.VectorSubcoreMesh(core_axis_name="c", subcore_axis_name="s", num_cores=_NC, num_subcores=_NS)
    return pl.kernel(
        functools.partial(_scatter_body, bpc=bpc),
        out_type=jax.ShapeDtypeStruct((nb * N, H), jnp.float32),
        mesh=mesh,
        scratch_types=[
            pltpu.VMEM_SHARED((bpc * N, H), jnp.float32),
            pltpu.VMEM((_CH,), jnp.int32),
            pltpu.VMEM((_CH,), jnp.int32),
            pltpu.VMEM((_CH, H), jnp.float32),
            pltpu.VMEM((_CH, H), jnp.float32),
            pltpu.VMEM((_CH, H), jnp.float32),
            pltpu.SemaphoreType.DMA,
            pltpu.SemaphoreType.DMA,
            pltpu.SemaphoreType.DMA,
            pltpu.SemaphoreType.DMA,
        ],
    )(m, dstl)


def _proj_body(nf, w, out):
    out[...] = jnp.dot(nf[...], w[0], preferred_element_type=jnp.float32)


def _tc_proj(nf, wstack):
    rows = nf.shape[0]
    return pl.pallas_call(
        _proj_body,
        grid=(2,),
        in_specs=[
            pl.BlockSpec((rows, H), lambda i: (0, 0)),
            pl.BlockSpec((1, H, H), lambda i: (i, 0, 0)),
        ],
        out_specs=pl.BlockSpec((rows, H), lambda i: (i, 0)),
        out_shape=jax.ShapeDtypeStruct((2 * rows, H), jnp.float32),
    )(nf, wstack)


_EBLK = 2048


def _edge_mlp_body(gsd, ef, we, b1, w2, b2, out):
    h = (
        gsd[...]
        + jnp.dot(ef[...], we[...], preferred_element_type=jnp.float32)
        + b1[...]
    )
    h = jnp.maximum(h, 0.0)
    h = jnp.dot(h, w2[...], preferred_element_type=jnp.float32) + b2[...]
    out[...] = jnp.maximum(h, 0.0)


def _tc_edge_mlp(gsd, ef, we, b1, w2, b2):
    rows = gsd.shape[0]
    grid = rows // _EBLK
    full = lambda shape: pl.BlockSpec(shape, lambda i: (0, 0))
    return pl.pallas_call(
        _edge_mlp_body,
        grid=(grid,),
        in_specs=[
            pl.BlockSpec((_EBLK, H), lambda i: (i, 0)),
            pl.BlockSpec((_EBLK, DE), lambda i: (i, 0)),
            full((DE, H)),
            full((1, H)),
            full((H, H)),
            full((1, H)),
        ],
        out_specs=pl.BlockSpec((_EBLK, H), lambda i: (i, 0)),
        out_shape=jax.ShapeDtypeStruct((rows, H), jnp.float32),
    )(gsd, ef, we, b1, w2, b2)


_NBLK = 2048


def _node_mlp_body(nf, agg, prev, wa, wb, b1, w2, b2, out, *, add_prev):
    h = (
        jnp.dot(nf[...], wa[...], preferred_element_type=jnp.float32)
        + jnp.dot(agg[...], wb[...], preferred_element_type=jnp.float32)
        + b1[...]
    )
    h = jnp.maximum(h, 0.0)
    h = jnp.dot(h, w2[...], preferred_element_type=jnp.float32) + b2[...]
    h = jnp.maximum(h, 0.0)
    if add_prev:
        h = h + prev[...]
    out[...] = h


def _tc_node_mlp(nf, agg, prev, wa, wb, b1, w2, b2, add_prev):
    rows = nf.shape[0]
    grid = rows // _NBLK
    full = lambda shape: pl.BlockSpec(shape, lambda i: (0, 0))
    return pl.pallas_call(
        functools.partial(_node_mlp_body, add_prev=add_prev),
        grid=(grid,),
        in_specs=[
            pl.BlockSpec((_NBLK, H), lambda i: (i, 0)),
            pl.BlockSpec((_NBLK, H), lambda i: (i, 0)),
            pl.BlockSpec((_NBLK, H), lambda i: (i, 0)),
            full((H, H)),
            full((H, H)),
            full((1, H)),
            full((H, H)),
            full((1, H)),
        ],
        out_specs=pl.BlockSpec((_NBLK, H), lambda i: (i, 0)),
        out_shape=jax.ShapeDtypeStruct((rows, H), jnp.float32),
    )(nf, agg, prev, wa, wb, b1, w2, b2)


def _attn_body(nf, scale, ws, bs, wh, bh, wzm, bzm, wlv, blv, zm, lv):
    sc = scale[0, 0]
    pooled_rows = []
    for b in range(B):
        x = nf[b]
        scores = sc * lax.dot_general(
            x, x, (((1,), (1,)), ((), ())), preferred_element_type=jnp.float32
        )
        rowmax = jnp.max(scores, axis=1, keepdims=True)
        ex = jnp.exp(scores - rowmax)
        rsum = jnp.sum(ex, axis=1, keepdims=True)
        colw = jnp.sum(ex / rsum, axis=0, keepdims=True) * (1.0 / N)
        pooled_rows.append(jnp.dot(colw, x, preferred_element_type=jnp.float32))
    pooled = jnp.concatenate(pooled_rows, axis=0)
    x1 = jnp.dot(pooled, ws[...], preferred_element_type=jnp.float32) + bs[...]
    x2 = jnp.dot(x1, wh[...], preferred_element_type=jnp.float32) + bh[...]
    x2 = jnp.maximum(x2, 0.0)
    zm[...] = jnp.dot(x2, wzm[...], preferred_element_type=jnp.float32) + bzm[...]
    lv[...] = jnp.dot(x2, wlv[...], preferred_element_type=jnp.float32) + blv[...]


def _tc_attn_head(nf, scale, ws, bs, wh, bh, wzm, bzm, wlv, blv):
    return pl.pallas_call(
        _attn_body,
        in_specs=[
            pl.BlockSpec((B, N, H), lambda: (0, 0, 0)),
            pl.BlockSpec(memory_space=pltpu.SMEM),
            pl.BlockSpec((H, H), lambda: (0, 0)),
            pl.BlockSpec((1, H), lambda: (0, 0)),
            pl.BlockSpec((H, HID2), lambda: (0, 0)),
            pl.BlockSpec((1, HID2), lambda: (0, 0)),
            pl.BlockSpec((HID2, H), lambda: (0, 0)),
            pl.BlockSpec((1, H), lambda: (0, 0)),
            pl.BlockSpec((HID2, H), lambda: (0, 0)),
            pl.BlockSpec((1, H), lambda: (0, 0)),
        ],
        out_specs=[
            pl.BlockSpec((B, H), lambda: (0, 0)),
            pl.BlockSpec((B, H), lambda: (0, 0)),
        ],
        out_shape=[
            jax.ShapeDtypeStruct((B, H), jnp.float32),
            jax.ShapeDtypeStruct((B, H), jnp.float32),
        ],
    )(nf, scale, ws, bs, wh, bh, wzm, bzm, wlv, blv)


def kernel(node_features, edge_features, edge_src, edge_dst, prop, params):
    ef = edge_features.reshape(B * E, DE)

    HB = B // 2
    hE, hN = HB * E, HB * N
    boff4 = (jnp.arange(HB, dtype=jnp.int32) * N)[:, None]
    lboff = ((jnp.arange(HB, dtype=jnp.int32) % 2) * N)[:, None]
    sx, dx, dl, efh = [], [], [], []
    for h in range(2):
        es = edge_src[h * HB : (h + 1) * HB]
        ed = edge_dst[h * HB : (h + 1) * HB]
        sx.append((es + boff4).reshape(-1))
        dx.append((ed + boff4 + hN).reshape(-1))
        dl.append((ed + lboff).reshape(-1))
        efh.append(ef[h * hE : (h + 1) * hE])

    nfs = [node_features.reshape(B * N, H)[h * hN : (h + 1) * hN] for h in range(2)]
    prevs = [None, None]
    for l in range(3):
        p = params["mp"][l]
        wstack = jnp.stack([p["Wm1"][:H], p["Wm1"][H : 2 * H]])
        we, bm1 = p["Wm1"][2 * H :], p["bm1"].reshape(1, H)
        wm2, bm2 = p["Wm2"], p["bm2"].reshape(1, H)
        for h in range(2):
            table = _tc_proj(nfs[h], wstack)
            gsd = _sc_gather(table, sx[h], dx[h])
            m = _tc_edge_mlp(gsd, efh[h], we, bm1, wm2, bm2)
            agg = _sc_scatter_add(m, dl[h])
            nf_new = _tc_node_mlp(
                nfs[h], agg, prevs[h] if prevs[h] is not None else nfs[h],
                p["Wu1"][:H], p["Wu1"][H:], p["bu1"].reshape(1, H),
                p["Wu2"], p["bu2"].reshape(1, H),
                add_prev=prevs[h] is not None,
            )
            prevs[h] = nf_new
            nfs[h] = nf_new
    nf = jnp.concatenate(nfs, axis=0)

    scale = params["attn_scale"].reshape(1, 1)
    wzm = jnp.pad(params["Wzm"], ((0, 0), (0, H - ZD)))
    bzm = jnp.pad(params["bzm"], (0, H - ZD)).reshape(1, H)
    wlv = jnp.pad(params["Wlv"], ((0, 0), (0, H - ZD)))
    blv = jnp.pad(params["blv"], (0, H - ZD)).reshape(1, H)
    zm_pad, lv_pad = _tc_attn_head(
        nf.reshape(B, N, H), scale,
        params["Ws2s"], params["bs2s"].reshape(1, H),
        params["Wh"], params["bh"].reshape(1, HID2),
        wzm, bzm, wlv, blv,
    )
    return (zm_pad[:, :ZD], lv_pad[:, :ZD])
